# baseline (speedup 1.0000x reference)
import functools

import jax
import jax.numpy as jnp
from jax import lax
from jax.experimental import pallas as pl
from jax.experimental.pallas import tpu as pltpu
from jax.experimental.pallas import tpu_sc as plsc

LANES = 16
SC_CHUNKS = 18


def _sc_add_pe(x2d, lead_table, time_table, c0, n_sc_chunks):
  info = plsc.get_sparse_core_info()
  nw = info.num_cores * info.num_subcores
  n_leads, d = lead_table.shape
  n_frames = time_table.shape[0]
  fpw = n_frames // nw
  nvec = d // LANES

  mesh = plsc.VectorSubcoreMesh(core_axis_name="c", subcore_axis_name="s")

  @functools.partial(
      pl.kernel,
      mesh=mesh,
      out_type=jax.ShapeDtypeStruct((n_sc_chunks * n_frames, d), jnp.float32),
      scratch_types=[
          pltpu.VMEM((fpw, d), jnp.float32),
          pltpu.VMEM((fpw, d), jnp.float32),
          pltpu.VMEM((fpw, d), jnp.float32),
          pltpu.VMEM((fpw, d), jnp.float32),
          pltpu.VMEM((fpw, d), jnp.float32),
          pltpu.VMEM((n_leads, d), jnp.float32),
          pltpu.SemaphoreType.DMA,
          pltpu.SemaphoreType.DMA,
          pltpu.SemaphoreType.DMA,
          pltpu.SemaphoreType.DMA,
      ],
  )
  def k(x_hbm, lead_hbm, time_hbm, out_hbm,
        xb0, xb1, yb0, yb1, tv, lv, si0, si1, so0, so1):
    w = lax.axis_index("s") * info.num_cores + lax.axis_index("c")
    f0 = w * fpw

    pltpu.sync_copy(time_hbm.at[pl.ds(f0, fpw), :], tv)
    pltpu.sync_copy(lead_hbm, lv)

    def in_copy(c, buf, sem):
      return pltpu.make_async_copy(
          x_hbm.at[pl.ds((c0 + c) * n_frames + f0, fpw), :], buf, sem)

    def out_copy(c, buf, sem):
      return pltpu.make_async_copy(
          buf, out_hbm.at[pl.ds(c * n_frames + f0, fpw), :], sem)

    def compute(c, xb, yb):
      l = lax.rem(c0 + c, n_leads)

      def kbody(kk, _):
        off = kk * LANES
        lvec = lv[l, pl.ds(off, LANES)]

        def rbody(r, carry):
          yb[r, pl.ds(off, LANES)] = (
              xb[r, pl.ds(off, LANES)] + tv[r, pl.ds(off, LANES)] + lvec)
          return carry

        return lax.fori_loop(0, fpw, rbody, _, unroll=4)

      lax.fori_loop(0, nvec, kbody, 0)

    in_copy(0, xb0, si0).start()
    in_copy(1, xb1, si1).start()

    def step(i, carry):
      for phase, (xb, yb, si, so) in enumerate(
          ((xb0, yb0, si0, so0), (xb1, yb1, si1, so1))):
        c = 2 * i + phase
        in_copy(c, xb, si).wait()

        @pl.when(i >= 1)
        def _():
          out_copy(c - 2, yb, so).wait()

        compute(c, xb, yb)
        out_copy(c, yb, so).start()

        @pl.when(c + 2 < n_sc_chunks)
        def _():
          in_copy(c + 2, xb, si).start()
      return carry

    lax.fori_loop(0, n_sc_chunks // 2, step, 0)
    out_copy(n_sc_chunks - 2, yb0, so0).wait()
    out_copy(n_sc_chunks - 1, yb1, so1).wait()

  return k(x2d, lead_table, time_table)


def _tc_add_pe(x3d, lead_table, time_table, n_tc_chunks):
  n_chunks, n_frames, d = x3d.shape
  n_leads = lead_table.shape[0]
  cpb = 2

  def body(x_ref, lead_ref, time_ref, o_ref):
    for j in range(cpb):
      l = lax.rem(pl.program_id(0) * cpb + j, n_leads)
      o_ref[j] = x_ref[j] + time_ref[...] + lead_ref[pl.ds(l, 1), :]

  return pl.pallas_call(
      body,
      grid=(n_tc_chunks // cpb,),
      in_specs=[
          pl.BlockSpec((cpb, n_frames, d), lambda i: (i, 0, 0)),
          pl.BlockSpec((n_leads, d), lambda i: (0, 0)),
          pl.BlockSpec((n_frames, d), lambda i: (0, 0)),
      ],
      out_specs=pl.BlockSpec((cpb, n_frames, d), lambda i: (i, 0, 0)),
      out_shape=jax.ShapeDtypeStruct((n_chunks, n_frames, d), jnp.float32),
  )(x3d, lead_table, time_table)


def kernel(x, lead_table, time_table, n_leads, n_frames):
  del n_leads, n_frames
  batch, seq, d = x.shape
  nl, nf = lead_table.shape[0], time_table.shape[0]
  x2d = x.reshape(batch * seq, d)
  n_chunks = x2d.shape[0] // nf
  c0 = n_chunks - SC_CHUNKS
  sc_out = _sc_add_pe(x2d, lead_table, time_table, c0, SC_CHUNKS)
  tc_out = _tc_add_pe(x2d.reshape(n_chunks, nf, d), lead_table, time_table, c0)
  out2d = lax.dynamic_update_slice(
      tc_out.reshape(batch * seq, d), sc_out, (c0 * nf, 0))
  return out2d.reshape(batch, seq, d)


# speedup vs baseline: 1.0533x; 1.0533x over previous
import functools

import jax
import jax.numpy as jnp
from jax import lax
from jax.experimental import pallas as pl
from jax.experimental.pallas import tpu as pltpu
from jax.experimental.pallas import tpu_sc as plsc

LANES = 16
SC_CHUNKS = 16


def _sc_add_pe(x2d, lead_table, time_table, c0, n_sc_chunks):
  info = plsc.get_sparse_core_info()
  nw = info.num_cores * info.num_subcores
  n_leads, d = lead_table.shape
  n_frames = time_table.shape[0]
  fpw = n_frames // nw
  nvec = d // LANES

  mesh = plsc.VectorSubcoreMesh(core_axis_name="c", subcore_axis_name="s")

  @functools.partial(
      pl.kernel,
      mesh=mesh,
      out_type=jax.ShapeDtypeStruct((n_sc_chunks * n_frames, d), jnp.float32),
      scratch_types=[
          pltpu.VMEM((fpw, d), jnp.float32),
          pltpu.VMEM((fpw, d), jnp.float32),
          pltpu.VMEM((fpw, d), jnp.float32),
          pltpu.VMEM((fpw, d), jnp.float32),
          pltpu.VMEM((fpw, d), jnp.float32),
          pltpu.VMEM((n_leads, d), jnp.float32),
          pltpu.SemaphoreType.DMA,
          pltpu.SemaphoreType.DMA,
          pltpu.SemaphoreType.DMA,
          pltpu.SemaphoreType.DMA,
      ],
  )
  def k(x_hbm, lead_hbm, time_hbm, out_hbm,
        xb0, xb1, yb0, yb1, tv, lv, si0, si1, so0, so1):
    w = lax.axis_index("s") * info.num_cores + lax.axis_index("c")
    f0 = w * fpw

    pltpu.sync_copy(time_hbm.at[pl.ds(f0, fpw), :], tv)
    pltpu.sync_copy(lead_hbm, lv)

    def in_copy(c, buf, sem):
      return pltpu.make_async_copy(
          x_hbm.at[pl.ds((c0 + c) * n_frames + f0, fpw), :], buf, sem)

    def out_copy(c, buf, sem):
      return pltpu.make_async_copy(
          buf, out_hbm.at[pl.ds(c * n_frames + f0, fpw), :], sem)

    def compute(c, xb, yb):
      l = lax.rem(c0 + c, n_leads)

      def kbody(kk, _):
        off = kk * LANES
        lvec = lv[l, pl.ds(off, LANES)]

        def rbody(r, carry):
          yb[r, pl.ds(off, LANES)] = (
              xb[r, pl.ds(off, LANES)] + tv[r, pl.ds(off, LANES)] + lvec)
          return carry

        return lax.fori_loop(0, fpw, rbody, _, unroll=4)

      lax.fori_loop(0, nvec, kbody, 0)

    in_copy(0, xb0, si0).start()
    in_copy(1, xb1, si1).start()

    def step(i, carry):
      for phase, (xb, yb, si, so) in enumerate(
          ((xb0, yb0, si0, so0), (xb1, yb1, si1, so1))):
        c = 2 * i + phase
        in_copy(c, xb, si).wait()

        @pl.when(i >= 1)
        def _():
          out_copy(c - 2, yb, so).wait()

        compute(c, xb, yb)
        out_copy(c, yb, so).start()

        @pl.when(c + 2 < n_sc_chunks)
        def _():
          in_copy(c + 2, xb, si).start()
      return carry

    lax.fori_loop(0, n_sc_chunks // 2, step, 0)
    out_copy(n_sc_chunks - 2, yb0, so0).wait()
    out_copy(n_sc_chunks - 1, yb1, so1).wait()

  return k(x2d, lead_table, time_table)


def _tc_add_pe(x3d, lead_table, time_table, n_tc_chunks):
  n_chunks, n_frames, d = x3d.shape
  n_leads = lead_table.shape[0]
  cpb = 4

  def body(x_ref, lead_ref, time_ref, o_ref):
    for j in range(cpb):
      l = lax.rem(pl.program_id(0) * cpb + j, n_leads)
      o_ref[j] = x_ref[j] + time_ref[...] + lead_ref[pl.ds(l, 1), :]

  return pl.pallas_call(
      body,
      grid=(n_tc_chunks // cpb,),
      in_specs=[
          pl.BlockSpec((cpb, n_frames, d), lambda i: (i, 0, 0)),
          pl.BlockSpec((n_leads, d), lambda i: (0, 0)),
          pl.BlockSpec((n_frames, d), lambda i: (0, 0)),
      ],
      out_specs=pl.BlockSpec((cpb, n_frames, d), lambda i: (i, 0, 0)),
      out_shape=jax.ShapeDtypeStruct((n_chunks, n_frames, d), jnp.float32),
  )(x3d, lead_table, time_table)


def kernel(x, lead_table, time_table, n_leads, n_frames):
  del n_leads, n_frames
  batch, seq, d = x.shape
  nl, nf = lead_table.shape[0], time_table.shape[0]
  x2d = x.reshape(batch * seq, d)
  n_chunks = x2d.shape[0] // nf
  c0 = n_chunks - SC_CHUNKS
  sc_out = _sc_add_pe(x2d, lead_table, time_table, c0, SC_CHUNKS)
  tc_out = _tc_add_pe(x2d.reshape(n_chunks, nf, d), lead_table, time_table, c0)
  out2d = lax.dynamic_update_slice(
      tc_out.reshape(batch * seq, d), sc_out, (c0 * nf, 0))
  return out2d.reshape(batch, seq, d)


# speedup vs baseline: 1.0546x; 1.0012x over previous
import functools

import jax
import jax.numpy as jnp
from jax import lax
from jax.experimental import pallas as pl
from jax.experimental.pallas import tpu as pltpu
from jax.experimental.pallas import tpu_sc as plsc

LANES = 16
SC_CHUNKS = 16


def _sc_add_pe(x2d, lead_table, time_table, c0, n_sc_chunks):
  info = plsc.get_sparse_core_info()
  nw = info.num_cores * info.num_subcores
  n_leads, d = lead_table.shape
  n_frames = time_table.shape[0]
  fpw = n_frames // nw
  nvec = d // LANES

  mesh = plsc.VectorSubcoreMesh(core_axis_name="c", subcore_axis_name="s")

  @functools.partial(
      pl.kernel,
      mesh=mesh,
      out_type=jax.ShapeDtypeStruct((n_sc_chunks * n_frames, d), jnp.float32),
      scratch_types=[
          pltpu.VMEM((fpw, d), jnp.float32),
          pltpu.VMEM((fpw, d), jnp.float32),
          pltpu.VMEM((fpw, d), jnp.float32),
          pltpu.VMEM((fpw, d), jnp.float32),
          pltpu.VMEM((fpw, d), jnp.float32),
          pltpu.VMEM((n_leads, d), jnp.float32),
          pltpu.SemaphoreType.DMA,
          pltpu.SemaphoreType.DMA,
          pltpu.SemaphoreType.DMA,
          pltpu.SemaphoreType.DMA,
      ],
  )
  def k(x_hbm, lead_hbm, time_hbm, out_hbm,
        xb0, xb1, yb0, yb1, tv, lv, si0, si1, so0, so1):
    w = lax.axis_index("s") * info.num_cores + lax.axis_index("c")
    f0 = w * fpw

    pltpu.sync_copy(time_hbm.at[pl.ds(f0, fpw), :], tv)
    pltpu.sync_copy(lead_hbm, lv)

    def in_copy(c, buf, sem):
      return pltpu.make_async_copy(
          x_hbm.at[pl.ds((c0 + c) * n_frames + f0, fpw), :], buf, sem)

    def out_copy(c, buf, sem):
      return pltpu.make_async_copy(
          buf, out_hbm.at[pl.ds(c * n_frames + f0, fpw), :], sem)

    def compute(c, xb, yb):
      l = lax.rem(c0 + c, n_leads)

      def kbody(kk, _):
        off = kk * LANES
        lvec = lv[l, pl.ds(off, LANES)]

        def rbody(r, carry):
          yb[r, pl.ds(off, LANES)] = (
              xb[r, pl.ds(off, LANES)] + tv[r, pl.ds(off, LANES)] + lvec)
          return carry

        return lax.fori_loop(0, fpw, rbody, _, unroll=4)

      lax.fori_loop(0, nvec, kbody, 0)

    in_copy(0, xb0, si0).start()
    in_copy(1, xb1, si1).start()

    def step(i, carry):
      for phase, (xb, yb, si, so) in enumerate(
          ((xb0, yb0, si0, so0), (xb1, yb1, si1, so1))):
        c = 2 * i + phase
        in_copy(c, xb, si).wait()

        @pl.when(i >= 1)
        def _():
          out_copy(c - 2, yb, so).wait()

        compute(c, xb, yb)
        out_copy(c, yb, so).start()

        @pl.when(c + 2 < n_sc_chunks)
        def _():
          in_copy(c + 2, xb, si).start()
      return carry

    lax.fori_loop(0, n_sc_chunks // 2, step, 0)
    out_copy(n_sc_chunks - 2, yb0, so0).wait()
    out_copy(n_sc_chunks - 1, yb1, so1).wait()

  return k(x2d, lead_table, time_table)


def _tc_add_pe(x3d, lead_table, time_table, n_tc_chunks):
  n_chunks, n_frames, d = x3d.shape
  n_leads = lead_table.shape[0]
  cpb = 2

  def body(x_ref, lead_ref, time_ref, o_ref):
    for j in range(cpb):
      l = lax.rem(pl.program_id(0) * cpb + j, n_leads)
      o_ref[j] = x_ref[j] + time_ref[...] + lead_ref[pl.ds(l, 1), :]

  return pl.pallas_call(
      body,
      grid=(n_tc_chunks // cpb,),
      in_specs=[
          pl.BlockSpec((cpb, n_frames, d), lambda i: (i, 0, 0)),
          pl.BlockSpec((n_leads, d), lambda i: (0, 0)),
          pl.BlockSpec((n_frames, d), lambda i: (0, 0)),
      ],
      out_specs=pl.BlockSpec((cpb, n_frames, d), lambda i: (i, 0, 0)),
      out_shape=jax.ShapeDtypeStruct((n_chunks, n_frames, d), jnp.float32),
  )(x3d, lead_table, time_table)


def kernel(x, lead_table, time_table, n_leads, n_frames):
  del n_leads, n_frames
  batch, seq, d = x.shape
  nf = time_table.shape[0]
  x2d = x.reshape(batch * seq, d)
  n_chunks = x2d.shape[0] // nf
  c0 = n_chunks - SC_CHUNKS
  sc_out = _sc_add_pe(x2d, lead_table, time_table, c0, SC_CHUNKS)
  tc_out = _tc_add_pe(x2d.reshape(n_chunks, nf, d), lead_table, time_table, c0)
  out2d = lax.dynamic_update_slice(
      tc_out.reshape(batch * seq, d), sc_out, (c0 * nf, 0))
  return out2d.reshape(batch, seq, d)


# speedup vs baseline: 1.0610x; 1.0061x over previous
import functools

import jax
import jax.numpy as jnp
from jax import lax
from jax.experimental import pallas as pl
from jax.experimental.pallas import tpu as pltpu
from jax.experimental.pallas import tpu_sc as plsc

LANES = 16
SC_CHUNKS = 16


def _sc_add_pe(x2d, lead_table, time_table, c0, n_sc_chunks):
  info = plsc.get_sparse_core_info()
  nw = info.num_cores * info.num_subcores
  n_leads, d = lead_table.shape
  n_frames = time_table.shape[0]
  fpw = n_frames // nw
  nvec = d // LANES

  mesh = plsc.VectorSubcoreMesh(core_axis_name="c", subcore_axis_name="s")

  @functools.partial(
      pl.kernel,
      mesh=mesh,
      out_type=jax.ShapeDtypeStruct((n_sc_chunks * n_frames, d), jnp.float32),
      scratch_types=[
          pltpu.VMEM((fpw, d), jnp.float32),
          pltpu.VMEM((fpw, d), jnp.float32),
          pltpu.VMEM((fpw, d), jnp.float32),
          pltpu.VMEM((fpw, d), jnp.float32),
          pltpu.VMEM((fpw, d), jnp.float32),
          pltpu.VMEM((n_leads, d), jnp.float32),
          pltpu.SemaphoreType.DMA,
          pltpu.SemaphoreType.DMA,
          pltpu.SemaphoreType.DMA,
          pltpu.SemaphoreType.DMA,
      ],
  )
  def k(x_hbm, lead_hbm, time_hbm, out_hbm,
        xb0, xb1, yb0, yb1, tv, lv, si0, si1, so0, so1):
    w = lax.axis_index("s") * info.num_cores + lax.axis_index("c")
    f0 = w * fpw

    pltpu.sync_copy(time_hbm.at[pl.ds(f0, fpw), :], tv)
    pltpu.sync_copy(lead_hbm, lv)

    def in_copy(c, buf, sem):
      return pltpu.make_async_copy(
          x_hbm.at[pl.ds((c0 + c) * n_frames + f0, fpw), :], buf, sem)

    def out_copy(c, buf, sem):
      return pltpu.make_async_copy(
          buf, out_hbm.at[pl.ds(c * n_frames + f0, fpw), :], sem)

    def compute(c, xb, yb):
      l = lax.rem(c0 + c, n_leads)

      def kbody(kk, _):
        off = kk * LANES
        lvec = lv[l, pl.ds(off, LANES)]

        def rbody(r, carry):
          yb[r, pl.ds(off, LANES)] = (
              xb[r, pl.ds(off, LANES)] + tv[r, pl.ds(off, LANES)] + lvec)
          return carry

        return lax.fori_loop(0, fpw, rbody, _, unroll=4)

      lax.fori_loop(0, nvec, kbody, 0)

    in_copy(0, xb0, si0).start()
    in_copy(1, xb1, si1).start()

    def step(i, carry):
      for phase, (xb, yb, si, so) in enumerate(
          ((xb0, yb0, si0, so0), (xb1, yb1, si1, so1))):
        c = 2 * i + phase
        in_copy(c, xb, si).wait()

        @pl.when(i >= 1)
        def _():
          out_copy(c - 2, yb, so).wait()

        compute(c, xb, yb)
        out_copy(c, yb, so).start()

        @pl.when(c + 2 < n_sc_chunks)
        def _():
          in_copy(c + 2, xb, si).start()
      return carry

    lax.fori_loop(0, n_sc_chunks // 2, step, 0)
    out_copy(n_sc_chunks - 2, yb0, so0).wait()
    out_copy(n_sc_chunks - 1, yb1, so1).wait()

  return k(x2d, lead_table, time_table)


def _tc_add_pe(x3d, lead_table, time_table, n_tc_chunks):
  n_chunks, n_frames, d = x3d.shape
  n_leads = lead_table.shape[0]
  cpb = 2

  def body(x_ref, lead_ref, time_ref, o_ref):
    for j in range(cpb):
      l = lax.rem(pl.program_id(0) * cpb + j, n_leads)
      o_ref[j] = x_ref[j] + time_ref[...] + lead_ref[pl.ds(l, 1), :]

  return pl.pallas_call(
      body,
      grid=(n_tc_chunks // cpb,),
      in_specs=[
          pl.BlockSpec((cpb, n_frames, d), lambda i: (i, 0, 0)),
          pl.BlockSpec((n_leads, d), lambda i: (0, 0)),
          pl.BlockSpec((n_frames, d), lambda i: (0, 0)),
      ],
      out_specs=pl.BlockSpec((cpb, n_frames, d), lambda i: (i, 0, 0)),
      out_shape=jax.ShapeDtypeStruct((n_chunks, n_frames, d), jnp.float32),
  )(x3d, lead_table, time_table)


def kernel(x, lead_table, time_table, n_leads, n_frames):
  del n_leads, n_frames
  batch, seq, d = x.shape
  nf = time_table.shape[0]
  x2d = x.reshape(batch * seq, d)
  n_chunks = x2d.shape[0] // nf
  c0 = n_chunks - SC_CHUNKS
  tc_out = _tc_add_pe(x2d.reshape(n_chunks, nf, d), lead_table, time_table, c0)
  sc_out = _sc_add_pe(x2d, lead_table, time_table, c0, SC_CHUNKS)
  out2d = lax.dynamic_update_slice(
      tc_out.reshape(batch * seq, d), sc_out, (c0 * nf, 0))
  return out2d.reshape(batch, seq, d)
